# baseline (speedup 1.0000x reference)
import numpy as np
import jax
import jax.numpy as jnp
from jax.experimental import pallas as pl
from jax.experimental.pallas import tpu as pltpu

_H = 64
_HP = 2 * _H
_L = 6
_B = 2
_N = 200
_NP = _N // 2
_TI = 40
_CPB = _N // _TI
_CH = _TI * _NP

_f32 = jnp.float32
_bf16 = jnp.bfloat16


def _mm(a, w, out=_f32):
    return jax.lax.dot_general(
        a.astype(_bf16), w.astype(_bf16),
        dimension_numbers=(((1,), (0,)), ((), ())),
        preferred_element_type=out)


def _mmb(a, w):
    return jax.lax.dot_general(
        a, w, dimension_numbers=(((1,), (0,)), ((), ())),
        preferred_element_type=_f32)


def _ln(v):
    m = jnp.mean(v, axis=-1, keepdims=True)
    s = jnp.mean((v - m) ** 2, axis=-1, keepdims=True)
    return (v - m) * jax.lax.rsqrt(s + 1e-5)


def _body(adj_ref, eps_ref, coords_ref, t_ref, nw_ref, nb_ref, ew_ref,
          eb_ref, tw1_ref, tb1_ref, tw2_ref, tb2_ref, u2_ref, m2_ref,
          V_ref, W_ref, A_ref, Bm_ref, C_ref, Tp_ref, ow_ref, ob_ref,
          out_ref, e_s, x_s, ag_s, vx_s, wx_s, cx_s, temb_s, tadd_s):
    c0 = coords_ref[:, 0:1]
    c1 = coords_ref[:, 1:2]
    x_s[...] = c0 * nw_ref[0:1, :] + c1 * nw_ref[1:2, :] + nb_ref[...]

    half = _H // 2
    j = jax.lax.broadcasted_iota(jnp.int32, (1, half), 1).astype(_f32)
    freqs = jnp.exp(-(np.log(10000.0) / half) * j)
    for b in range(_B):
        args = t_ref[b] * 1000.0 * freqs
        emb = jnp.concatenate([jnp.sin(args), jnp.cos(args)], axis=1)
        h1 = jnp.maximum(_mm(emb, tw1_ref[...]) + tb1_ref[...], 0.0)
        temb_s[pl.ds(b, 1), :] = _mm(h1, tw2_ref[...]) + tb2_ref[...]

    ew3 = ew_ref[...].reshape(1, 1, _H)
    eb3 = eb_ref[...].reshape(1, 1, _H)
    ow3 = ow_ref[...].reshape(1, 1, _H)
    ob = ob_ref[0]
    m2 = m2_ref[...]

    loss = _f32(0.0)
    for l in range(_L):
        xv = x_s[...]
        vx_s[...] = _mm(xv, V_ref[l])
        wx_s[...] = _mm(xv, W_ref[l])
        cx_s[...] = _mm(xv, C_ref[l])
        tadd_s[...] = _mm(temb_s[...], Tp_ref[l])
        u2 = u2_ref[l]

        for b in range(_B):
            tt = t_ref[b]
            td = tadd_s[b:b + 1, :]
            wxtd_f = wx_s[b * _N:(b + 1) * _N, :] + td
            wxtd = jnp.concatenate(
                [wxtd_f[:_NP, :], wxtd_f[_NP:, :]], axis=1)[None, :, :]
            cx_f = cx_s[b * _N:(b + 1) * _N, :]
            cxp = jnp.concatenate(
                [cx_f[:_NP, :], cx_f[_NP:, :]], axis=1)[None, :, :]

            def chunk(c, acc, l=l, b=b, tt=tt, wxtd=wxtd, cxp=cxp, u2=u2):
                if l == 0:
                    a = adj_ref[pl.ds(b * _N + c * _TI, _TI), :]
                    ep = eps_ref[pl.ds(b * _N + c * _TI, _TI), :]
                    adjt = (1.0 - tt) * (a * 2.0 - 1.0) + tt * ep
                    e0 = adjt[:, :, None] * ew3 + eb3
                    ech = jnp.concatenate(
                        [e0[:, :_NP, :], e0[:, _NP:, :]],
                        axis=2).reshape(_CH, _HP)
                    ech_bf = ech.astype(_bf16)
                else:
                    ech_bf = e_s[pl.ds(b * _N * _NP + c * _CH, _CH), :]
                    ech = ech_bf.astype(_f32)
                eU = _mmb(ech_bf, u2)
                vx = vx_s[pl.ds(b * _N + c * _TI, _TI), :]
                vxp = jnp.concatenate([vx, vx], axis=1)
                en = (eU.reshape(_TI, _NP, _HP) + vxp[:, None, :] + wxtd)
                g = 1.0 / (1.0 + jnp.exp(-en))
                nump = jnp.sum(g * cxp, axis=1)
                denp = jnp.sum(g, axis=1)
                num = nump[:, :_H] + nump[:, _H:]
                den = denp[:, :_H] + denp[:, _H:]
                ag_s[pl.ds(b * _N + c * _TI, _TI), :] = num / (den + 1e-6)
                en2 = en.reshape(_CH, _HP)
                m = _mmb(en2.astype(_bf16), m2)
                d = en2 - m
                s = _mmb((d * d).astype(_bf16), m2)
                enew = ech + jnp.maximum(d * jax.lax.rsqrt(s + 1e-5), 0.0)
                if l < _L - 1:
                    e_s[pl.ds(b * _N * _NP + c * _CH, _CH), :] = (
                        enew.astype(_bf16))
                    return acc
                e3 = enew.reshape(_TI, _NP, _HP)
                pvl = jnp.sum(e3[:, :, :_H] * ow3, axis=-1)
                pvr = jnp.sum(e3[:, :, _H:] * ow3, axis=-1)
                pv = jnp.concatenate([pvl, pvr], axis=1) + ob
                a = adj_ref[pl.ds(b * _N + c * _TI, _TI), :]
                ep = eps_ref[pl.ds(b * _N + c * _TI, _TI), :]
                df = pv - (ep - (a * 2.0 - 1.0))
                return acc + jnp.sum(df * df)

            loss = jax.lax.fori_loop(0, _CPB, chunk, loss)
        if l < _L - 1:
            xa = _mm(x_s[...], A_ref[l]) + _mm(ag_s[...], Bm_ref[l])
            x_s[...] = x_s[...] + jnp.maximum(_ln(xa), 0.0)

    out_ref[...] = (loss * (1.0 / (_B * _N * _N))).reshape(1, 1)


def kernel(coords, adj_0, t, epsilon, node_w, node_b, edge_w, edge_b, tw1,
           tb1, tw2, tb2, U, V, W, A, Bm, C, Tp, out_w, out_b):
    z64 = jnp.zeros((_L, _H, _H), _f32)
    u2 = jnp.concatenate([
        jnp.concatenate([U, z64], axis=2),
        jnp.concatenate([z64, U], axis=2)], axis=1).astype(_bf16)
    jm = jnp.full((_H, _H), 1.0 / _H, _f32)
    zm = jnp.zeros((_H, _H), _f32)
    m2 = jnp.concatenate([
        jnp.concatenate([jm, zm], axis=1),
        jnp.concatenate([zm, jm], axis=1)], axis=0).astype(_bf16)
    vmem = pl.BlockSpec(memory_space=pltpu.VMEM)
    smem = pl.BlockSpec(memory_space=pltpu.SMEM)
    out = pl.pallas_call(
        _body,
        out_shape=jax.ShapeDtypeStruct((1, 1), _f32),
        in_specs=[vmem, vmem, vmem, smem, vmem, vmem, vmem, vmem, vmem,
                  vmem, vmem, vmem, vmem, vmem, vmem, vmem, vmem, vmem,
                  vmem, vmem, vmem, smem],
        out_specs=vmem,
        scratch_shapes=[
            pltpu.VMEM((_B * _N * _NP, _HP), _bf16),
            pltpu.VMEM((_B * _N, _H), _f32),
            pltpu.VMEM((_B * _N, _H), _f32),
            pltpu.VMEM((_B * _N, _H), _f32),
            pltpu.VMEM((_B * _N, _H), _f32),
            pltpu.VMEM((_B * _N, _H), _f32),
            pltpu.VMEM((_B, _H), _f32),
            pltpu.VMEM((_B, _H), _f32),
        ],
    )(adj_0.reshape(_B * _N, _N), epsilon.reshape(_B * _N, _N),
      coords.reshape(_B * _N, 2), t, node_w, node_b.reshape(1, _H),
      edge_w, edge_b.reshape(1, _H), tw1, tb1.reshape(1, _H), tw2,
      tb2.reshape(1, _H), u2, m2, V, W, A, Bm, C, Tp, out_w.reshape(1, _H),
      out_b)
    return out.reshape(())


# speedup vs baseline: 1.0762x; 1.0762x over previous
import numpy as np
import jax
import jax.numpy as jnp
from jax.experimental import pallas as pl
from jax.experimental.pallas import tpu as pltpu

_H = 64
_HP = 2 * _H
_L = 6
_B = 2
_N = 200
_NP = _N // 2
_TI = 40
_CPB = _N // _TI
_CH = _TI * _NP

_f32 = jnp.float32
_bf16 = jnp.bfloat16


def _mm(a, w, out=_f32):
    return jax.lax.dot_general(
        a.astype(_bf16), w.astype(_bf16),
        dimension_numbers=(((1,), (0,)), ((), ())),
        preferred_element_type=out)


def _mmb(a, w):
    return jax.lax.dot_general(
        a, w, dimension_numbers=(((1,), (0,)), ((), ())),
        preferred_element_type=_f32)


def _ln(v):
    m = jnp.mean(v, axis=-1, keepdims=True)
    s = jnp.mean((v - m) ** 2, axis=-1, keepdims=True)
    return (v - m) * jax.lax.rsqrt(s + 1e-5)


def _pair(v):
    return jnp.concatenate([v[:_NP, :], v[_NP:, :]], axis=1)


def _body(adj_ref, eps_ref, coords_ref, t_ref, nw_ref, nb_ref, ew_ref,
          eb_ref, tw1_ref, tb1_ref, tw2_ref, tb2_ref, u2_ref, m2_ref,
          V_ref, W_ref, A_ref, Bm_ref, C_ref, Tp_ref, ow_ref, ob_ref,
          out_ref, e_s, x_s, ag_s, vx_s, wx_s, cx_s, temb_s, tadd_s):
    c0 = coords_ref[:, 0:1]
    c1 = coords_ref[:, 1:2]
    x_s[...] = c0 * nw_ref[0:1, :] + c1 * nw_ref[1:2, :] + nb_ref[...]

    half = _H // 2
    j = jax.lax.broadcasted_iota(jnp.int32, (1, half), 1).astype(_f32)
    freqs = jnp.exp(-(np.log(10000.0) / half) * j)
    for b in range(_B):
        args = t_ref[b] * 1000.0 * freqs
        emb = jnp.concatenate([jnp.sin(args), jnp.cos(args)], axis=1)
        h1 = jnp.maximum(_mm(emb, tw1_ref[...]) + tb1_ref[...], 0.0)
        temb_s[pl.ds(b, 1), :] = _mm(h1, tw2_ref[...]) + tb2_ref[...]

    ew3 = ew_ref[...].reshape(1, 1, _H)
    eb3 = eb_ref[...].reshape(1, 1, _H)
    tt0 = t_ref[0]
    tt1 = t_ref[1]

    def init_c(c, carry):
        for b, tt in ((0, tt0), (1, tt1)):
            a = adj_ref[pl.ds(b * _N + c * _TI, _TI), :]
            ep = eps_ref[pl.ds(b * _N + c * _TI, _TI), :]
            adjt = (1.0 - tt) * (a * 2.0 - 1.0) + tt * ep
            e0 = adjt[:, :, None] * ew3 + eb3
            e0p = jnp.concatenate([e0[:, :_NP, :], e0[:, _NP:, :]], axis=2)
            e_s[pl.ds(b * _N * _NP + c * _CH, _CH), :] = (
                e0p.reshape(_CH, _HP).astype(_bf16))
        return carry

    jax.lax.fori_loop(0, _CPB, init_c, 0)

    m2 = m2_ref[...]
    for l in range(_L):
        xv = x_s[...]
        vx_s[...] = _mm(xv, V_ref[l])
        wx_s[...] = _mm(xv, W_ref[l])
        cx_s[...] = _mm(xv, C_ref[l])
        tadd_s[...] = _mm(temb_s[...], Tp_ref[l])
        u2 = u2_ref[l]

        wxtd0 = _pair(wx_s[0:_N, :] + tadd_s[0:1, :])[None, :, :]
        wxtd1 = _pair(wx_s[_N:2 * _N, :] + tadd_s[1:2, :])[None, :, :]
        cxp0 = _pair(cx_s[0:_N, :])[None, :, :]
        cxp1 = _pair(cx_s[_N:2 * _N, :])[None, :, :]

        def chunk(c, carry, wxtd0=wxtd0, wxtd1=wxtd1, cxp0=cxp0,
                  cxp1=cxp1, u2=u2):
            for b, wxtd, cxp in ((0, wxtd0, cxp0), (1, wxtd1, cxp1)):
                ech_bf = e_s[pl.ds(b * _N * _NP + c * _CH, _CH), :]
                eU = _mmb(ech_bf, u2)
                vx = vx_s[pl.ds(b * _N + c * _TI, _TI), :]
                vxp = jnp.concatenate([vx, vx], axis=1)
                en = (eU.reshape(_TI, _NP, _HP) + vxp[:, None, :] + wxtd)
                g = 1.0 / (1.0 + jnp.exp(-en))
                nump = jnp.sum(g * cxp, axis=1)
                denp = jnp.sum(g, axis=1)
                num = nump[:, :_H] + nump[:, _H:]
                den = denp[:, :_H] + denp[:, _H:]
                ag_s[pl.ds(b * _N + c * _TI, _TI), :] = num / (den + 1e-6)
                en2 = en.reshape(_CH, _HP)
                m = _mmb(en2.astype(_bf16), m2)
                d = en2 - m
                s = _mmb((d * d).astype(_bf16), m2)
                enew = ech_bf.astype(_f32) + jnp.maximum(
                    d * jax.lax.rsqrt(s + 1e-5), 0.0)
                e_s[pl.ds(b * _N * _NP + c * _CH, _CH), :] = (
                    enew.astype(_bf16))
            return carry

        jax.lax.fori_loop(0, _CPB, chunk, 0)
        xa = _mm(x_s[...], A_ref[l]) + _mm(ag_s[...], Bm_ref[l])
        x_s[...] = x_s[...] + jnp.maximum(_ln(xa), 0.0)

    ow3 = ow_ref[...].reshape(1, 1, _H).astype(_bf16)
    ob = ob_ref[0]

    def loss_c(c, acc):
        for b in range(_B):
            e3 = e_s[pl.ds(b * _N * _NP + c * _CH, _CH), :].reshape(
                _TI, _NP, _HP)
            pvl = jnp.sum(e3[:, :, :_H] * ow3, axis=-1).astype(_f32)
            pvr = jnp.sum(e3[:, :, _H:] * ow3, axis=-1).astype(_f32)
            pv = jnp.concatenate([pvl, pvr], axis=1) + ob
            a = adj_ref[pl.ds(b * _N + c * _TI, _TI), :]
            ep = eps_ref[pl.ds(b * _N + c * _TI, _TI), :]
            df = pv - (ep - (a * 2.0 - 1.0))
            acc = acc + jnp.sum(df * df)
        return acc

    loss = jax.lax.fori_loop(0, _CPB, loss_c, _f32(0.0))
    out_ref[...] = (loss * (1.0 / (_B * _N * _N))).reshape(1, 1)


def kernel(coords, adj_0, t, epsilon, node_w, node_b, edge_w, edge_b, tw1,
           tb1, tw2, tb2, U, V, W, A, Bm, C, Tp, out_w, out_b):
    z64 = jnp.zeros((_L, _H, _H), _f32)
    u2 = jnp.concatenate([
        jnp.concatenate([U, z64], axis=2),
        jnp.concatenate([z64, U], axis=2)], axis=1).astype(_bf16)
    jm = jnp.full((_H, _H), 1.0 / _H, _f32)
    zm = jnp.zeros((_H, _H), _f32)
    m2 = jnp.concatenate([
        jnp.concatenate([jm, zm], axis=1),
        jnp.concatenate([zm, jm], axis=1)], axis=0).astype(_bf16)
    vmem = pl.BlockSpec(memory_space=pltpu.VMEM)
    smem = pl.BlockSpec(memory_space=pltpu.SMEM)
    out = pl.pallas_call(
        _body,
        out_shape=jax.ShapeDtypeStruct((1, 1), _f32),
        in_specs=[vmem, vmem, vmem, smem, vmem, vmem, vmem, vmem, vmem,
                  vmem, vmem, vmem, vmem, vmem, vmem, vmem, vmem, vmem,
                  vmem, vmem, vmem, smem],
        out_specs=vmem,
        scratch_shapes=[
            pltpu.VMEM((_B * _N * _NP, _HP), _bf16),
            pltpu.VMEM((_B * _N, _H), _f32),
            pltpu.VMEM((_B * _N, _H), _f32),
            pltpu.VMEM((_B * _N, _H), _f32),
            pltpu.VMEM((_B * _N, _H), _f32),
            pltpu.VMEM((_B * _N, _H), _f32),
            pltpu.VMEM((_B, _H), _f32),
            pltpu.VMEM((_B, _H), _f32),
        ],
    )(adj_0.reshape(_B * _N, _N), epsilon.reshape(_B * _N, _N),
      coords.reshape(_B * _N, 2), t, node_w, node_b.reshape(1, _H),
      edge_w, edge_b.reshape(1, _H), tw1, tb1.reshape(1, _H), tw2,
      tb2.reshape(1, _H), u2, m2, V, W, A, Bm, C, Tp, out_w.reshape(1, _H),
      out_b)
    return out.reshape(())


# speedup vs baseline: 1.1138x; 1.0349x over previous
import numpy as np
import jax
import jax.numpy as jnp
from jax.experimental import pallas as pl
from jax.experimental.pallas import tpu as pltpu

_H = 64
_HP = 2 * _H
_L = 6
_B = 2
_N = 200
_NP = _N // 2
_TI = 40
_CPB = _N // _TI
_CH = _TI * _NP

_f32 = jnp.float32
_bf16 = jnp.bfloat16


def _mm(a, w, out=_f32):
    return jax.lax.dot_general(
        a.astype(_bf16), w.astype(_bf16),
        dimension_numbers=(((1,), (0,)), ((), ())),
        preferred_element_type=out)


def _mmb(a, w):
    return jax.lax.dot_general(
        a, w, dimension_numbers=(((1,), (0,)), ((), ())),
        preferred_element_type=_f32)


def _ln(v):
    m = jnp.mean(v, axis=-1, keepdims=True)
    s = jnp.mean((v - m) ** 2, axis=-1, keepdims=True)
    return (v - m) * jax.lax.rsqrt(s + 1e-5)


def _pair(v):
    return jnp.concatenate([v[:_NP, :], v[_NP:, :]], axis=1)


def _body(adj_ref, eps_ref, coords_ref, t_ref, nw_ref, nb_ref, ew_ref,
          eb_ref, tw1_ref, tb1_ref, tw2_ref, tb2_ref, u2_ref, m2_ref,
          V_ref, W_ref, A_ref, Bm_ref, C_ref, Tp_ref, ow_ref, ob_ref,
          out_ref, e_s, x_s, ag_s, vx_s, wx_s, cx_s, temb_s, tadd_s):
    c0 = coords_ref[:, 0:1]
    c1 = coords_ref[:, 1:2]
    x_s[...] = c0 * nw_ref[0:1, :] + c1 * nw_ref[1:2, :] + nb_ref[...]

    half = _H // 2
    j = jax.lax.broadcasted_iota(jnp.int32, (1, half), 1).astype(_f32)
    freqs = jnp.exp(-(np.log(10000.0) / half) * j)
    for b in range(_B):
        args = t_ref[b] * 1000.0 * freqs
        emb = jnp.concatenate([jnp.sin(args), jnp.cos(args)], axis=1)
        h1 = jnp.maximum(_mm(emb, tw1_ref[...]) + tb1_ref[...], 0.0)
        temb_s[pl.ds(b, 1), :] = _mm(h1, tw2_ref[...]) + tb2_ref[...]

    ew3 = ew_ref[...].reshape(1, 1, _H)
    eb3 = eb_ref[...].reshape(1, 1, _H)
    tt0 = t_ref[0]
    tt1 = t_ref[1]

    def init_c(c, carry):
        for b, tt in ((0, tt0), (1, tt1)):
            a = adj_ref[pl.ds(b * _N + c * _TI, _TI), :]
            ep = eps_ref[pl.ds(b * _N + c * _TI, _TI), :]
            adjt = (1.0 - tt) * (a * 2.0 - 1.0) + tt * ep
            e0 = adjt[:, :, None] * ew3 + eb3
            e0p = jnp.concatenate([e0[:, :_NP, :], e0[:, _NP:, :]], axis=2)
            e_s[pl.ds(b * _N * _NP + c * _CH, _CH), :] = (
                e0p.reshape(_CH, _HP).astype(_bf16))
        return carry

    jax.lax.fori_loop(0, _CPB, init_c, 0)

    m2 = m2_ref[...]
    for l in range(_L):
        xv = x_s[...]
        vx_s[...] = _mm(xv, V_ref[l])
        wx_s[...] = _mm(xv, W_ref[l])
        cx_s[...] = _mm(xv, C_ref[l])
        tadd_s[...] = _mm(temb_s[...], Tp_ref[l])
        u2 = u2_ref[l]

        wxtd0 = _pair(wx_s[0:_N, :] + tadd_s[0:1, :])[None, :, :]
        wxtd1 = _pair(wx_s[_N:2 * _N, :] + tadd_s[1:2, :])[None, :, :]
        cxp0 = _pair(cx_s[0:_N, :])[None, :, :]
        cxp1 = _pair(cx_s[_N:2 * _N, :])[None, :, :]

        def chunk(c, carry, wxtd0=wxtd0, wxtd1=wxtd1, cxp0=cxp0,
                  cxp1=cxp1, u2=u2):
            for b, wxtd, cxp in ((0, wxtd0, cxp0), (1, wxtd1, cxp1)):
                ech_bf = e_s[pl.ds(b * _N * _NP + c * _CH, _CH), :]
                eU = _mmb(ech_bf, u2)
                vx = vx_s[pl.ds(b * _N + c * _TI, _TI), :]
                vxp = jnp.concatenate([vx, vx], axis=1)
                en = (eU.reshape(_TI, _NP, _HP) + vxp[:, None, :] + wxtd)
                g = jax.nn.sigmoid(en)
                nump = jnp.sum(g * cxp, axis=1)
                denp = jnp.sum(g, axis=1)
                num = nump[:, :_H] + nump[:, _H:]
                den = denp[:, :_H] + denp[:, _H:]
                ag_s[pl.ds(b * _N + c * _TI, _TI), :] = num / (den + 1e-6)
                en2 = en.reshape(_CH, _HP)
                m = _mmb(en2.astype(_bf16), m2)
                d = en2 - m
                s = _mmb((d * d).astype(_bf16), m2)
                enew = ech_bf.astype(_f32) + jnp.maximum(
                    d * jax.lax.rsqrt(s + 1e-5), 0.0)
                e_s[pl.ds(b * _N * _NP + c * _CH, _CH), :] = (
                    enew.astype(_bf16))
            return carry

        jax.lax.fori_loop(0, _CPB, chunk, 0)
        xa = _mm(x_s[...], A_ref[l]) + _mm(ag_s[...], Bm_ref[l])
        x_s[...] = x_s[...] + jnp.maximum(_ln(xa), 0.0)

    ow3 = ow_ref[...].reshape(1, 1, _H).astype(_bf16)
    ob = ob_ref[0]

    def loss_c(c, acc):
        for b in range(_B):
            e3 = e_s[pl.ds(b * _N * _NP + c * _CH, _CH), :].reshape(
                _TI, _NP, _HP)
            pvl = jnp.sum(e3[:, :, :_H] * ow3, axis=-1).astype(_f32)
            pvr = jnp.sum(e3[:, :, _H:] * ow3, axis=-1).astype(_f32)
            pv = jnp.concatenate([pvl, pvr], axis=1) + ob
            a = adj_ref[pl.ds(b * _N + c * _TI, _TI), :]
            ep = eps_ref[pl.ds(b * _N + c * _TI, _TI), :]
            df = pv - (ep - (a * 2.0 - 1.0))
            acc = acc + jnp.sum(df * df)
        return acc

    loss = jax.lax.fori_loop(0, _CPB, loss_c, _f32(0.0))
    out_ref[...] = (loss * (1.0 / (_B * _N * _N))).reshape(1, 1)


def kernel(coords, adj_0, t, epsilon, node_w, node_b, edge_w, edge_b, tw1,
           tb1, tw2, tb2, U, V, W, A, Bm, C, Tp, out_w, out_b):
    z64 = jnp.zeros((_L, _H, _H), _f32)
    u2 = jnp.concatenate([
        jnp.concatenate([U, z64], axis=2),
        jnp.concatenate([z64, U], axis=2)], axis=1).astype(_bf16)
    jm = jnp.full((_H, _H), 1.0 / _H, _f32)
    zm = jnp.zeros((_H, _H), _f32)
    m2 = jnp.concatenate([
        jnp.concatenate([jm, zm], axis=1),
        jnp.concatenate([zm, jm], axis=1)], axis=0).astype(_bf16)
    vmem = pl.BlockSpec(memory_space=pltpu.VMEM)
    smem = pl.BlockSpec(memory_space=pltpu.SMEM)
    out = pl.pallas_call(
        _body,
        out_shape=jax.ShapeDtypeStruct((1, 1), _f32),
        in_specs=[vmem, vmem, vmem, smem, vmem, vmem, vmem, vmem, vmem,
                  vmem, vmem, vmem, vmem, vmem, vmem, vmem, vmem, vmem,
                  vmem, vmem, vmem, smem],
        out_specs=vmem,
        scratch_shapes=[
            pltpu.VMEM((_B * _N * _NP, _HP), _bf16),
            pltpu.VMEM((_B * _N, _H), _f32),
            pltpu.VMEM((_B * _N, _H), _f32),
            pltpu.VMEM((_B * _N, _H), _f32),
            pltpu.VMEM((_B * _N, _H), _f32),
            pltpu.VMEM((_B * _N, _H), _f32),
            pltpu.VMEM((_B, _H), _f32),
            pltpu.VMEM((_B, _H), _f32),
        ],
    )(adj_0.reshape(_B * _N, _N), epsilon.reshape(_B * _N, _N),
      coords.reshape(_B * _N, 2), t, node_w, node_b.reshape(1, _H),
      edge_w, edge_b.reshape(1, _H), tw1, tb1.reshape(1, _H), tw2,
      tb2.reshape(1, _H), u2, m2, V, W, A, Bm, C, Tp, out_w.reshape(1, _H),
      out_b)
    return out.reshape(())
